# baseline (speedup 1.0000x reference)
import functools

import jax
import jax.numpy as jnp
from jax import lax
from jax.experimental import pallas as pl
from jax.experimental.pallas import tpu as pltpu
from jax.experimental.pallas import tpu_sc as plsc

N_NODES = 10000
N_EDGES = 320000
NS = 16
C = N_EDGES // NS
LANES = 16
NCH = 10
CHW = C // NCH
CV = CHW // LANES
WU = 5
NU = 5
ZCH = 640
ZCH_LAST = N_NODES - 15 * ZCH


def _build():
  mesh = plsc.VectorSubcoreMesh(core_axis_name="c", subcore_axis_name="s")

  @functools.partial(
      pl.kernel,
      mesh=mesh,
      out_type=jax.ShapeDtypeStruct((2 * N_EDGES,), jnp.float32),
      scratch_types=(
          [pltpu.VMEM((CHW,), jnp.int32) for _ in range(NCH)]
          + [pltpu.VMEM((CHW,), jnp.int32) for _ in range(NCH)]
          + [pltpu.VMEM((CHW,), jnp.float32) for _ in range(NCH)]
          + [pltpu.VMEM((CHW,), jnp.float32) for _ in range(NCH)]
          + [
              pltpu.VMEM((3 * N_NODES,), jnp.float32),
              pltpu.VMEM((N_NODES,), jnp.float32),
              pltpu.VMEM_SHARED((N_NODES,), jnp.float32),
              pltpu.SemaphoreType.DMA,
              pltpu.SemaphoreType.DMA,
              pltpu.SemaphoreType.DMA,
          ]
          + [pltpu.SemaphoreType.DMA for _ in range(NCH)]
      ),
      compiler_params=pltpu.CompilerParams(needs_layout_passes=False),
  )
  def vm_kernel(ef_hbm, verts_hbm, ovals, *refs):
    key = refs[0:NCH]
    oth = refs[NCH:2 * NCH]
    w = refs[2 * NCH:3 * NCH]
    sv = refs[3 * NCH:4 * NCH]
    vt, invv, ws_sh, sem_in, sem_sc, sem_out = refs[4 * NCH:4 * NCH + 6]
    sem_st = refs[4 * NCH + 6:]

    cid = lax.axis_index("c")
    sid = lax.axis_index("s")
    base = sid * C
    kbase = cid * N_EDGES + base
    obase = (1 - cid) * N_EDGES + base

    h_vt = pltpu.async_copy(verts_hbm, vt, sem_in)
    stage = []
    for k in range(NCH):
      stage.append((
          pltpu.async_copy(ef_hbm.at[pl.ds(kbase + k * CHW, CHW)], key[k],
                           sem_st[k]),
          pltpu.async_copy(ef_hbm.at[pl.ds(obase + k * CHW, CHW)], oth[k],
                           sem_st[k]),
      ))

    zeros = jnp.zeros((LANES,), jnp.float32)

    def zbody(i, carry):
      sv[0][pl.ds(i * LANES, LANES)] = zeros
      return carry

    @pl.when(sid < NS - 1)
    def _():
      lax.fori_loop(0, ZCH // LANES, zbody, 0)
      pltpu.sync_copy(sv[0].at[pl.ds(0, ZCH)], ws_sh.at[pl.ds(sid * ZCH, ZCH)])

    @pl.when(sid == NS - 1)
    def _():
      lax.fori_loop(0, ZCH_LAST // LANES, zbody, 0)
      pltpu.sync_copy(sv[0].at[pl.ds(0, ZCH_LAST)],
                      ws_sh.at[pl.ds((NS - 1) * ZCH, ZCH_LAST)])

    plsc.subcore_barrier()
    h_vt.wait()

    scat = []
    for k in range(NCH):
      kk, ok, wk = key[k], oth[k], w[k]
      stage[k][0].wait()
      stage[k][1].wait()

      def wbody(i, carry, kk=kk, ok=ok, wk=wk):
        for u in range(WU):
          s = pl.ds(i * (WU * LANES) + u * LANES, LANES)
          i0 = kk[s] * 3
          i1 = ok[s] * 3
          dx = plsc.load_gather(vt, [i1]) - plsc.load_gather(vt, [i0])
          dy = plsc.load_gather(vt, [i1 + 1]) - plsc.load_gather(vt, [i0 + 1])
          dz = plsc.load_gather(vt, [i1 + 2]) - plsc.load_gather(vt, [i0 + 2])
          wk[s] = 1.0 / (1.0 + dx * dx + dy * dy + dz * dz)
        return carry

      lax.fori_loop(0, CV // WU, wbody, 0)
      scat.append(pltpu.async_copy(wk, ws_sh.at[kk], sem_sc, add=True))

    for h in scat:
      h.wait()
    plsc.subcore_barrier()

    def rbody(i, carry):
      s = pl.ds(i * LANES, LANES)
      sv[0][s] = 0.5 / sv[0][s]
      return carry

    @pl.when(sid < NS - 1)
    def _():
      pltpu.sync_copy(ws_sh.at[pl.ds(sid * ZCH, ZCH)], sv[0].at[pl.ds(0, ZCH)])
      lax.fori_loop(0, ZCH // LANES, rbody, 0)
      pltpu.sync_copy(sv[0].at[pl.ds(0, ZCH)], ws_sh.at[pl.ds(sid * ZCH, ZCH)])

    @pl.when(sid == NS - 1)
    def _():
      pltpu.sync_copy(ws_sh.at[pl.ds((NS - 1) * ZCH, ZCH_LAST)],
                      sv[0].at[pl.ds(0, ZCH_LAST)])
      lax.fori_loop(0, ZCH_LAST // LANES, rbody, 0)
      pltpu.sync_copy(sv[0].at[pl.ds(0, ZCH_LAST)],
                      ws_sh.at[pl.ds((NS - 1) * ZCH, ZCH_LAST)])

    plsc.subcore_barrier()

    pltpu.sync_copy(ws_sh, invv)

    outs = []
    for k in range(NCH):
      kk, wk, svk = key[k], w[k], sv[k]

      def nbody(i, carry, kk=kk, wk=wk, svk=svk):
        for u in range(NU):
          s = pl.ds(i * (NU * LANES) + u * LANES, LANES)
          svk[s] = wk[s] * plsc.load_gather(invv, [kk[s]])
        return carry

      lax.fori_loop(0, CV // NU, nbody, 0)
      outs.append(pltpu.async_copy(
          svk, ovals.at[pl.ds(kbase + k * CHW, CHW)], sem_out))

    for h in outs:
      h.wait()

  return vm_kernel


_VM_KERNEL = _build()


@jax.jit
def kernel(features, vertices, edges, faces):
  del features, faces
  ef = edges.reshape(2 * N_EDGES)
  out_vals = _VM_KERNEL(ef, vertices.reshape(3 * N_NODES))
  out_idx = jnp.stack([ef, jnp.roll(ef, -N_EDGES)])
  return out_idx, out_vals


# speedup vs baseline: 1.0079x; 1.0079x over previous
import functools

import jax
import jax.numpy as jnp
from jax import lax
from jax.experimental import pallas as pl
from jax.experimental.pallas import tpu as pltpu
from jax.experimental.pallas import tpu_sc as plsc

N_NODES = 10000
N_EDGES = 320000
NS = 16
C = N_EDGES // NS
LANES = 16
NCH = 5
CHW = C // NCH
CV = CHW // LANES
WU = 5
NU = 5
ZCH = 640
ZCH_LAST = N_NODES - 15 * ZCH


def _build():
  mesh = plsc.VectorSubcoreMesh(core_axis_name="c", subcore_axis_name="s")

  @functools.partial(
      pl.kernel,
      mesh=mesh,
      out_type=jax.ShapeDtypeStruct((2 * N_EDGES,), jnp.float32),
      scratch_types=(
          [pltpu.VMEM((CHW,), jnp.int32) for _ in range(NCH)]
          + [pltpu.VMEM((CHW,), jnp.int32) for _ in range(NCH)]
          + [pltpu.VMEM((CHW,), jnp.float32) for _ in range(NCH)]
          + [pltpu.VMEM((CHW,), jnp.float32) for _ in range(NCH)]
          + [
              pltpu.VMEM((3 * N_NODES,), jnp.float32),
              pltpu.VMEM((N_NODES,), jnp.float32),
              pltpu.VMEM_SHARED((N_NODES,), jnp.float32),
              pltpu.SemaphoreType.DMA,
              pltpu.SemaphoreType.DMA,
              pltpu.SemaphoreType.DMA,
          ]
          + [pltpu.SemaphoreType.DMA for _ in range(NCH)]
      ),
      compiler_params=pltpu.CompilerParams(needs_layout_passes=False),
  )
  def vm_kernel(ef_hbm, verts_hbm, ovals, *refs):
    key = refs[0:NCH]
    oth = refs[NCH:2 * NCH]
    w = refs[2 * NCH:3 * NCH]
    sv = refs[3 * NCH:4 * NCH]
    vt, invv, ws_sh, sem_in, sem_sc, sem_out = refs[4 * NCH:4 * NCH + 6]
    sem_st = refs[4 * NCH + 6:]

    cid = lax.axis_index("c")
    sid = lax.axis_index("s")
    base = sid * C
    kbase = cid * N_EDGES + base
    obase = (1 - cid) * N_EDGES + base

    h_vt = pltpu.async_copy(verts_hbm, vt, sem_in)
    stage = []
    for k in range(NCH):
      stage.append((
          pltpu.async_copy(ef_hbm.at[pl.ds(kbase + k * CHW, CHW)], key[k],
                           sem_st[k]),
          pltpu.async_copy(ef_hbm.at[pl.ds(obase + k * CHW, CHW)], oth[k],
                           sem_st[k]),
      ))

    zeros = jnp.zeros((LANES,), jnp.float32)

    def zbody(i, carry):
      sv[0][pl.ds(i * LANES, LANES)] = zeros
      return carry

    @pl.when(sid < NS - 1)
    def _():
      lax.fori_loop(0, ZCH // LANES, zbody, 0)
      pltpu.sync_copy(sv[0].at[pl.ds(0, ZCH)], ws_sh.at[pl.ds(sid * ZCH, ZCH)])

    @pl.when(sid == NS - 1)
    def _():
      lax.fori_loop(0, ZCH_LAST // LANES, zbody, 0)
      pltpu.sync_copy(sv[0].at[pl.ds(0, ZCH_LAST)],
                      ws_sh.at[pl.ds((NS - 1) * ZCH, ZCH_LAST)])

    plsc.subcore_barrier()
    h_vt.wait()

    scat = []
    for k in range(NCH):
      kk, ok, wk = key[k], oth[k], w[k]
      stage[k][0].wait()
      stage[k][1].wait()

      def wbody(i, carry, kk=kk, ok=ok, wk=wk):
        for u in range(WU):
          s = pl.ds(i * (WU * LANES) + u * LANES, LANES)
          i0 = kk[s] * 3
          i1 = ok[s] * 3
          dx = plsc.load_gather(vt, [i1]) - plsc.load_gather(vt, [i0])
          dy = plsc.load_gather(vt, [i1 + 1]) - plsc.load_gather(vt, [i0 + 1])
          dz = plsc.load_gather(vt, [i1 + 2]) - plsc.load_gather(vt, [i0 + 2])
          wk[s] = 1.0 / (1.0 + dx * dx + dy * dy + dz * dz)
        return carry

      lax.fori_loop(0, CV // WU, wbody, 0)
      scat.append(pltpu.async_copy(wk, ws_sh.at[kk], sem_sc, add=True))

    for h in scat:
      h.wait()
    plsc.subcore_barrier()

    def rbody(i, carry):
      s = pl.ds(i * LANES, LANES)
      sv[0][s] = 0.5 / sv[0][s]
      return carry

    @pl.when(sid < NS - 1)
    def _():
      pltpu.sync_copy(ws_sh.at[pl.ds(sid * ZCH, ZCH)], sv[0].at[pl.ds(0, ZCH)])
      lax.fori_loop(0, ZCH // LANES, rbody, 0)
      pltpu.sync_copy(sv[0].at[pl.ds(0, ZCH)], ws_sh.at[pl.ds(sid * ZCH, ZCH)])

    @pl.when(sid == NS - 1)
    def _():
      pltpu.sync_copy(ws_sh.at[pl.ds((NS - 1) * ZCH, ZCH_LAST)],
                      sv[0].at[pl.ds(0, ZCH_LAST)])
      lax.fori_loop(0, ZCH_LAST // LANES, rbody, 0)
      pltpu.sync_copy(sv[0].at[pl.ds(0, ZCH_LAST)],
                      ws_sh.at[pl.ds((NS - 1) * ZCH, ZCH_LAST)])

    plsc.subcore_barrier()

    pltpu.sync_copy(ws_sh, invv)

    outs = []
    for k in range(NCH):
      kk, wk, svk = key[k], w[k], sv[k]

      def nbody(i, carry, kk=kk, wk=wk, svk=svk):
        for u in range(NU):
          s = pl.ds(i * (NU * LANES) + u * LANES, LANES)
          svk[s] = wk[s] * plsc.load_gather(invv, [kk[s]])
        return carry

      lax.fori_loop(0, CV // NU, nbody, 0)
      outs.append(pltpu.async_copy(
          svk, ovals.at[pl.ds(kbase + k * CHW, CHW)], sem_out))

    for h in outs:
      h.wait()

  return vm_kernel


_VM_KERNEL = _build()


@jax.jit
def kernel(features, vertices, edges, faces):
  del features, faces
  ef = edges.reshape(2 * N_EDGES)
  out_vals = _VM_KERNEL(ef, vertices.reshape(3 * N_NODES))
  out_idx = jnp.stack([ef, jnp.roll(ef, -N_EDGES)])
  return out_idx, out_vals


# speedup vs baseline: 1.0082x; 1.0003x over previous
import functools

import jax
import jax.numpy as jnp
from jax import lax
from jax.experimental import pallas as pl
from jax.experimental.pallas import tpu as pltpu
from jax.experimental.pallas import tpu_sc as plsc

N_NODES = 10000
N_EDGES = 320000
NS = 16
C = N_EDGES // NS
LANES = 16
NCH = 5
CHW = C // NCH
CV = CHW // LANES
WU = 5
NU = 5
ZCH = 640
ZCH_LAST = N_NODES - 15 * ZCH


def _build():
  mesh = plsc.VectorSubcoreMesh(core_axis_name="c", subcore_axis_name="s")

  @functools.partial(
      pl.kernel,
      mesh=mesh,
      out_type=jax.ShapeDtypeStruct((2 * N_EDGES,), jnp.float32),
      scratch_types=(
          [pltpu.VMEM((CHW,), jnp.int32) for _ in range(NCH)]
          + [pltpu.VMEM((CHW,), jnp.int32) for _ in range(NCH)]
          + [pltpu.VMEM((CHW,), jnp.float32) for _ in range(NCH)]
          + [pltpu.VMEM((CHW,), jnp.float32) for _ in range(NCH)]
          + [
              pltpu.VMEM((3 * N_NODES,), jnp.float32),
              pltpu.VMEM((N_NODES,), jnp.float32),
              pltpu.VMEM_SHARED((N_NODES,), jnp.float32),
              pltpu.SemaphoreType.DMA,
              pltpu.SemaphoreType.DMA,
              pltpu.SemaphoreType.DMA,
          ]
          + [pltpu.SemaphoreType.DMA for _ in range(NCH)]
      ),
      compiler_params=pltpu.CompilerParams(needs_layout_passes=False),
  )
  def vm_kernel(ef_hbm, verts_hbm, ovals, *refs):
    key = refs[0:NCH]
    oth = refs[NCH:2 * NCH]
    w = refs[2 * NCH:3 * NCH]
    sv = refs[3 * NCH:4 * NCH]
    vt, invv, ws_sh, sem_in, sem_sc, sem_out = refs[4 * NCH:4 * NCH + 6]
    sem_st = refs[4 * NCH + 6:]

    cid = lax.axis_index("c")
    sid = lax.axis_index("s")
    base = sid * C
    kbase = cid * N_EDGES + base
    obase = (1 - cid) * N_EDGES + base

    h_vt = pltpu.async_copy(verts_hbm, vt, sem_in)
    stage = []
    for k in range(NCH):
      stage.append((
          pltpu.async_copy(ef_hbm.at[pl.ds(kbase + k * CHW, CHW)], key[k],
                           sem_st[k]),
          pltpu.async_copy(ef_hbm.at[pl.ds(obase + k * CHW, CHW)], oth[k],
                           sem_st[k]),
      ))

    zeros = jnp.zeros((LANES,), jnp.float32)

    def zbody(i, carry):
      sv[0][pl.ds(i * LANES, LANES)] = zeros
      return carry

    @pl.when(sid < NS - 1)
    def _():
      lax.fori_loop(0, ZCH // LANES, zbody, 0)
      pltpu.sync_copy(sv[0].at[pl.ds(0, ZCH)], ws_sh.at[pl.ds(sid * ZCH, ZCH)])

    @pl.when(sid == NS - 1)
    def _():
      lax.fori_loop(0, ZCH_LAST // LANES, zbody, 0)
      pltpu.sync_copy(sv[0].at[pl.ds(0, ZCH_LAST)],
                      ws_sh.at[pl.ds((NS - 1) * ZCH, ZCH_LAST)])

    with jax.named_scope("ph1_zero_barrier"):
      plsc.subcore_barrier()
    with jax.named_scope("ph2_vt_wait"):
      h_vt.wait()

    scat = []
    for k in range(NCH):
      kk, ok, wk = key[k], oth[k], w[k]
      with jax.named_scope("ph3_stage_wait"):
        stage[k][0].wait()
        stage[k][1].wait()

      def wbody(i, carry, kk=kk, ok=ok, wk=wk):
        for u in range(WU):
          s = pl.ds(i * (WU * LANES) + u * LANES, LANES)
          i0 = kk[s] * 3
          i1 = ok[s] * 3
          dx = plsc.load_gather(vt, [i1]) - plsc.load_gather(vt, [i0])
          dy = plsc.load_gather(vt, [i1 + 1]) - plsc.load_gather(vt, [i0 + 1])
          dz = plsc.load_gather(vt, [i1 + 2]) - plsc.load_gather(vt, [i0 + 2])
          wk[s] = 1.0 / (1.0 + dx * dx + dy * dy + dz * dz)
        return carry

      lax.fori_loop(0, CV // WU, wbody, 0)
      scat.append(pltpu.async_copy(wk, ws_sh.at[kk], sem_sc, add=True))

    with jax.named_scope("ph4_scat_drain"):
      for h in scat:
        h.wait()
      plsc.subcore_barrier()

    def rbody(i, carry):
      s = pl.ds(i * LANES, LANES)
      sv[0][s] = 0.5 / sv[0][s]
      return carry

    @pl.when(sid < NS - 1)
    def _():
      pltpu.sync_copy(ws_sh.at[pl.ds(sid * ZCH, ZCH)], sv[0].at[pl.ds(0, ZCH)])
      lax.fori_loop(0, ZCH // LANES, rbody, 0)
      pltpu.sync_copy(sv[0].at[pl.ds(0, ZCH)], ws_sh.at[pl.ds(sid * ZCH, ZCH)])

    @pl.when(sid == NS - 1)
    def _():
      pltpu.sync_copy(ws_sh.at[pl.ds((NS - 1) * ZCH, ZCH_LAST)],
                      sv[0].at[pl.ds(0, ZCH_LAST)])
      lax.fori_loop(0, ZCH_LAST // LANES, rbody, 0)
      pltpu.sync_copy(sv[0].at[pl.ds(0, ZCH_LAST)],
                      ws_sh.at[pl.ds((NS - 1) * ZCH, ZCH_LAST)])

    plsc.subcore_barrier()

    with jax.named_scope("ph6_invv_copy"):
      pltpu.sync_copy(ws_sh, invv)

    outs = []
    for k in range(NCH):
      kk, wk, svk = key[k], w[k], sv[k]

      def nbody(i, carry, kk=kk, wk=wk, svk=svk):
        for u in range(NU):
          s = pl.ds(i * (NU * LANES) + u * LANES, LANES)
          svk[s] = wk[s] * plsc.load_gather(invv, [kk[s]])
        return carry

      lax.fori_loop(0, CV // NU, nbody, 0)
      outs.append(pltpu.async_copy(
          svk, ovals.at[pl.ds(kbase + k * CHW, CHW)], sem_out))

    with jax.named_scope("ph8_out_drain"):
      for h in outs:
        h.wait()

  return vm_kernel


_VM_KERNEL = _build()


@jax.jit
def kernel(features, vertices, edges, faces):
  del features, faces
  ef = edges.reshape(2 * N_EDGES)
  out_vals = _VM_KERNEL(ef, vertices.reshape(3 * N_NODES))
  out_idx = jnp.stack([ef, jnp.roll(ef, -N_EDGES)])
  return out_idx, out_vals


# speedup vs baseline: 1.0182x; 1.0100x over previous
import functools

import jax
import jax.numpy as jnp
from jax import lax
from jax.experimental import pallas as pl
from jax.experimental.pallas import tpu as pltpu
from jax.experimental.pallas import tpu_sc as plsc

N_NODES = 10000
N_EDGES = 320000
NS = 16
C = N_EDGES // NS
LANES = 16
NCH = 5
CHW = C // NCH
CV = CHW // LANES
WU = 5
NU = 5
ZCH = 640
ZCH_LAST = N_NODES - 15 * ZCH
VP = 16
VTP = 30720
VPW = VTP // VP


def _build():
  mesh = plsc.VectorSubcoreMesh(core_axis_name="c", subcore_axis_name="s")

  @functools.partial(
      pl.kernel,
      mesh=mesh,
      out_type=jax.ShapeDtypeStruct((2 * N_EDGES,), jnp.float32),
      scratch_types=(
          [pltpu.VMEM((CHW,), jnp.int32) for _ in range(NCH)]
          + [pltpu.VMEM((CHW,), jnp.int32) for _ in range(NCH)]
          + [pltpu.VMEM((CHW,), jnp.float32) for _ in range(NCH)]
          + [pltpu.VMEM((CHW,), jnp.float32) for _ in range(NCH)]
          + [
              pltpu.VMEM((VTP,), jnp.float32),
              pltpu.VMEM((N_NODES,), jnp.float32),
              pltpu.VMEM_SHARED((N_NODES,), jnp.float32),
              pltpu.SemaphoreType.DMA,
              pltpu.SemaphoreType.DMA,
              pltpu.SemaphoreType.DMA,
          ]
          + [pltpu.SemaphoreType.DMA for _ in range(NCH)]
      ),
      compiler_params=pltpu.CompilerParams(needs_layout_passes=False),
  )
  def vm_kernel(ef_hbm, verts_hbm, ovals, *refs):
    key = refs[0:NCH]
    oth = refs[NCH:2 * NCH]
    w = refs[2 * NCH:3 * NCH]
    sv = refs[3 * NCH:4 * NCH]
    vt, invv, ws_sh, sem_in, sem_sc, sem_out = refs[4 * NCH:4 * NCH + 6]
    sem_st = refs[4 * NCH + 6:]

    cid = lax.axis_index("c")
    sid = lax.axis_index("s")
    base = sid * C
    kbase = cid * N_EDGES + base
    obase = (1 - cid) * N_EDGES + base

    h_vt = []
    for i in range(VP):
      pj = (sid + i) % VP
      voff = pj * VPW
      h_vt.append(pltpu.async_copy(
          verts_hbm.at[pl.ds(voff, VPW)], vt.at[pl.ds(voff, VPW)], sem_in))
    stage = []
    for k in range(NCH):
      stage.append((
          pltpu.async_copy(ef_hbm.at[pl.ds(kbase + k * CHW, CHW)], key[k],
                           sem_st[k]),
          pltpu.async_copy(ef_hbm.at[pl.ds(obase + k * CHW, CHW)], oth[k],
                           sem_st[k]),
      ))

    zeros = jnp.zeros((LANES,), jnp.float32)

    def zbody(i, carry):
      sv[0][pl.ds(i * LANES, LANES)] = zeros
      return carry

    @pl.when(sid < NS - 1)
    def _():
      lax.fori_loop(0, ZCH // LANES, zbody, 0)
      pltpu.sync_copy(sv[0].at[pl.ds(0, ZCH)], ws_sh.at[pl.ds(sid * ZCH, ZCH)])

    @pl.when(sid == NS - 1)
    def _():
      lax.fori_loop(0, ZCH_LAST // LANES, zbody, 0)
      pltpu.sync_copy(sv[0].at[pl.ds(0, ZCH_LAST)],
                      ws_sh.at[pl.ds((NS - 1) * ZCH, ZCH_LAST)])

    with jax.named_scope("ph1_zero_barrier"):
      plsc.subcore_barrier()
    with jax.named_scope("ph2_vt_wait"):
      for h in h_vt:
        h.wait()

    scat = []
    for k in range(NCH):
      kk, ok, wk = key[k], oth[k], w[k]
      with jax.named_scope("ph3_stage_wait"):
        stage[k][0].wait()
        stage[k][1].wait()

      def wbody(i, carry, kk=kk, ok=ok, wk=wk):
        for u in range(WU):
          s = pl.ds(i * (WU * LANES) + u * LANES, LANES)
          i0 = kk[s] * 3
          i1 = ok[s] * 3
          dx = plsc.load_gather(vt, [i1]) - plsc.load_gather(vt, [i0])
          dy = plsc.load_gather(vt, [i1 + 1]) - plsc.load_gather(vt, [i0 + 1])
          dz = plsc.load_gather(vt, [i1 + 2]) - plsc.load_gather(vt, [i0 + 2])
          wk[s] = 1.0 / (1.0 + dx * dx + dy * dy + dz * dz)
        return carry

      with jax.named_scope("ph3b_wloop"):
        lax.fori_loop(0, CV // WU, wbody, 0)
      scat.append(pltpu.async_copy(wk, ws_sh.at[kk], sem_sc, add=True))

    with jax.named_scope("ph4_scat_drain"):
      for h in scat:
        h.wait()
      plsc.subcore_barrier()

    def rbody(i, carry):
      s = pl.ds(i * LANES, LANES)
      sv[0][s] = 0.5 / sv[0][s]
      return carry

    @pl.when(sid < NS - 1)
    def _():
      pltpu.sync_copy(ws_sh.at[pl.ds(sid * ZCH, ZCH)], sv[0].at[pl.ds(0, ZCH)])
      lax.fori_loop(0, ZCH // LANES, rbody, 0)
      pltpu.sync_copy(sv[0].at[pl.ds(0, ZCH)], ws_sh.at[pl.ds(sid * ZCH, ZCH)])

    @pl.when(sid == NS - 1)
    def _():
      pltpu.sync_copy(ws_sh.at[pl.ds((NS - 1) * ZCH, ZCH_LAST)],
                      sv[0].at[pl.ds(0, ZCH_LAST)])
      lax.fori_loop(0, ZCH_LAST // LANES, rbody, 0)
      pltpu.sync_copy(sv[0].at[pl.ds(0, ZCH_LAST)],
                      ws_sh.at[pl.ds((NS - 1) * ZCH, ZCH_LAST)])

    with jax.named_scope("ph5_recip_barrier"):
      plsc.subcore_barrier()

    with jax.named_scope("ph6_invv_copy"):
      pltpu.sync_copy(ws_sh, invv)

    outs = []
    for k in range(NCH):
      kk, wk, svk = key[k], w[k], sv[k]

      def nbody(i, carry, kk=kk, wk=wk, svk=svk):
        for u in range(NU):
          s = pl.ds(i * (NU * LANES) + u * LANES, LANES)
          svk[s] = wk[s] * plsc.load_gather(invv, [kk[s]])
        return carry

      with jax.named_scope("ph7_nloop"):
        lax.fori_loop(0, CV // NU, nbody, 0)
      outs.append(pltpu.async_copy(
          svk, ovals.at[pl.ds(kbase + k * CHW, CHW)], sem_out))

    with jax.named_scope("ph8_out_drain"):
      for h in outs:
        h.wait()

  return vm_kernel


_VM_KERNEL = _build()


@jax.jit
def kernel(features, vertices, edges, faces):
  del features, faces
  ef = edges.reshape(2 * N_EDGES)
  vflat = jnp.concatenate(
      [vertices.reshape(3 * N_NODES),
       jnp.zeros((VTP - 3 * N_NODES,), jnp.float32)])
  out_vals = _VM_KERNEL(ef, vflat)
  out_idx = jnp.stack([ef, jnp.roll(ef, -N_EDGES)])
  return out_idx, out_vals


# speedup vs baseline: 1.2457x; 1.2234x over previous
import functools

import jax
import jax.numpy as jnp
from jax import lax
from jax.experimental import pallas as pl
from jax.experimental.pallas import tpu as pltpu
from jax.experimental.pallas import tpu_sc as plsc

N_NODES = 10000
N_EDGES = 320000
NS = 16
C = N_EDGES // NS
LANES = 16
NCH = 5
CHW = C // NCH
CV = CHW // LANES
WU = 5
NU = 5
ZCH = 640
ZCH_LAST = N_NODES - 15 * ZCH
VP = 16
VTP = 30720
VPW = VTP // VP


def _build():
  mesh = plsc.VectorSubcoreMesh(core_axis_name="c", subcore_axis_name="s")

  @functools.partial(
      pl.kernel,
      mesh=mesh,
      out_type=jax.ShapeDtypeStruct((2 * N_EDGES,), jnp.float32),
      scratch_types=(
          [pltpu.VMEM((CHW,), jnp.int32) for _ in range(NCH)]
          + [pltpu.VMEM((CHW,), jnp.int32) for _ in range(NCH)]
          + [pltpu.VMEM((CHW,), jnp.float32) for _ in range(NCH)]
          + [pltpu.VMEM((CHW,), jnp.float32) for _ in range(NCH)]
          + [
              pltpu.VMEM((VTP,), jnp.float32),
              pltpu.VMEM((N_NODES,), jnp.float32),
              pltpu.VMEM_SHARED((N_NODES,), jnp.float32),
              pltpu.SemaphoreType.DMA,
              pltpu.SemaphoreType.DMA,
              pltpu.SemaphoreType.DMA,
          ]
          + [pltpu.SemaphoreType.DMA for _ in range(NCH)]
      ),
      compiler_params=pltpu.CompilerParams(needs_layout_passes=False),
  )
  def vm_kernel(ef_hbm, verts_hbm, ovals, *refs):
    key = refs[0:NCH]
    oth = refs[NCH:2 * NCH]
    w = refs[2 * NCH:3 * NCH]
    sv = refs[3 * NCH:4 * NCH]
    vt, invv, ws_sh, sem_in, sem_sc, sem_out = refs[4 * NCH:4 * NCH + 6]
    sem_st = refs[4 * NCH + 6:]

    cid = lax.axis_index("c")
    sid = lax.axis_index("s")
    base = sid * C
    kbase = cid * N_EDGES + base
    obase = (1 - cid) * N_EDGES + base

    h_vt = []
    for i in range(VP):
      pj = (sid + i) % VP
      voff = pj * VPW
      h_vt.append(pltpu.async_copy(
          verts_hbm.at[pl.ds(voff, VPW)], vt.at[pl.ds(voff, VPW)], sem_in))
    stage = []
    for k in range(NCH):
      stage.append((
          pltpu.async_copy(ef_hbm.at[pl.ds(kbase + k * CHW, CHW)], key[k],
                           sem_st[k]),
          pltpu.async_copy(ef_hbm.at[pl.ds(obase + k * CHW, CHW)], oth[k],
                           sem_st[k]),
      ))

    zeros = jnp.zeros((LANES,), jnp.float32)

    def zbody(i, carry):
      sv[0][pl.ds(i * LANES, LANES)] = zeros
      return carry

    @pl.when(sid < NS - 1)
    def _():
      lax.fori_loop(0, ZCH // LANES, zbody, 0)
      pltpu.sync_copy(sv[0].at[pl.ds(0, ZCH)], ws_sh.at[pl.ds(sid * ZCH, ZCH)])

    @pl.when(sid == NS - 1)
    def _():
      lax.fori_loop(0, ZCH_LAST // LANES, zbody, 0)
      pltpu.sync_copy(sv[0].at[pl.ds(0, ZCH_LAST)],
                      ws_sh.at[pl.ds((NS - 1) * ZCH, ZCH_LAST)])

    with jax.named_scope("ph1_zero_barrier"):
      plsc.subcore_barrier()
    with jax.named_scope("ph2_vt_wait"):
      for h in h_vt:
        h.wait()

    scat = []
    for k in range(NCH):
      kk, ok, wk = key[k], oth[k], w[k]
      with jax.named_scope("ph3_stage_wait"):
        stage[k][0].wait()
        stage[k][1].wait()

      with jax.named_scope("ph3b_wloop"):
        @plsc.parallel_loop(0, CV, 1, unroll=WU)
        def _(i, kk=kk, ok=ok, wk=wk):
          s = pl.ds(i * LANES, LANES)
          i0 = kk[s] * 3
          i1 = ok[s] * 3
          dx = plsc.load_gather(vt, [i1]) - plsc.load_gather(vt, [i0])
          dy = plsc.load_gather(vt, [i1 + 1]) - plsc.load_gather(vt, [i0 + 1])
          dz = plsc.load_gather(vt, [i1 + 2]) - plsc.load_gather(vt, [i0 + 2])
          wk[s] = 1.0 / (1.0 + dx * dx + dy * dy + dz * dz)
      scat.append(pltpu.async_copy(wk, ws_sh.at[kk], sem_sc, add=True))

    with jax.named_scope("ph4_scat_drain"):
      for h in scat:
        h.wait()
      plsc.subcore_barrier()

    def rbody(i, carry):
      s = pl.ds(i * LANES, LANES)
      sv[0][s] = 0.5 / sv[0][s]
      return carry

    @pl.when(sid < NS - 1)
    def _():
      pltpu.sync_copy(ws_sh.at[pl.ds(sid * ZCH, ZCH)], sv[0].at[pl.ds(0, ZCH)])
      lax.fori_loop(0, ZCH // LANES, rbody, 0)
      pltpu.sync_copy(sv[0].at[pl.ds(0, ZCH)], ws_sh.at[pl.ds(sid * ZCH, ZCH)])

    @pl.when(sid == NS - 1)
    def _():
      pltpu.sync_copy(ws_sh.at[pl.ds((NS - 1) * ZCH, ZCH_LAST)],
                      sv[0].at[pl.ds(0, ZCH_LAST)])
      lax.fori_loop(0, ZCH_LAST // LANES, rbody, 0)
      pltpu.sync_copy(sv[0].at[pl.ds(0, ZCH_LAST)],
                      ws_sh.at[pl.ds((NS - 1) * ZCH, ZCH_LAST)])

    with jax.named_scope("ph5_recip_barrier"):
      plsc.subcore_barrier()

    with jax.named_scope("ph6_invv_copy"):
      pltpu.sync_copy(ws_sh, invv)

    outs = []
    for k in range(NCH):
      kk, wk, svk = key[k], w[k], sv[k]

      with jax.named_scope("ph7_nloop"):
        @plsc.parallel_loop(0, CV, 1, unroll=NU)
        def _(i, kk=kk, wk=wk, svk=svk):
          s = pl.ds(i * LANES, LANES)
          svk[s] = wk[s] * plsc.load_gather(invv, [kk[s]])
      outs.append(pltpu.async_copy(
          svk, ovals.at[pl.ds(kbase + k * CHW, CHW)], sem_out))

    with jax.named_scope("ph8_out_drain"):
      for h in outs:
        h.wait()

  return vm_kernel


_VM_KERNEL = _build()


@jax.jit
def kernel(features, vertices, edges, faces):
  del features, faces
  ef = edges.reshape(2 * N_EDGES)
  vflat = jnp.concatenate(
      [vertices.reshape(3 * N_NODES),
       jnp.zeros((VTP - 3 * N_NODES,), jnp.float32)])
  out_vals = _VM_KERNEL(ef, vflat)
  out_idx = jnp.stack([ef, jnp.roll(ef, -N_EDGES)])
  return out_idx, out_vals
